# baseline (speedup 1.0000x reference)
import functools

import jax
import jax.numpy as jnp
from jax import lax
from jax.experimental import pallas as pl
from jax.experimental.pallas import tpu as pltpu
from jax.experimental.pallas import tpu_sc as plsc

_D = 64
_SCALE = 8.0

_NC = 2
_NS = 16
_NW = _NC * _NS

_BLK = 128
_TS = 2
_SB = _TS * _BLK
_RING = 2


@functools.partial(jax.jit, static_argnames=("n_t", "n_b"))
def _embed_lookup(table, xts, n_t, n_b):
    n_super = n_t // _TS
    per_w = n_t * _BLK
    mesh = plsc.VectorSubcoreMesh(core_axis_name="c", subcore_axis_name="s")

    scratch = [pltpu.VMEM((per_w,), jnp.int32)]
    scratch += [pltpu.VMEM((_SB, _D), jnp.float32) for _ in range(_RING)]
    scratch += [pltpu.VMEM((_TS, _D // 8, 8, _BLK), jnp.float32)
                for _ in range(_RING)]
    scratch += [pltpu.SemaphoreType.DMA for _ in range(2 * _RING)]

    @functools.partial(
        pl.kernel,
        out_type=jax.ShapeDtypeStruct((n_t, _D // 8, _NW, 8, _BLK),
                                      jnp.float32),
        mesh=mesh,
        scratch_types=scratch,
        compiler_params=pltpu.CompilerParams(
            use_tc_tiling_on_sc=False, needs_layout_passes=False),
    )
    def k(table_hbm, xts_hbm, out_hbm, idx_all, *bufs):
        rows = bufs[0:_RING]
        trans = bufs[_RING:2 * _RING]
        gsem = bufs[2 * _RING:3 * _RING]
        wsem = bufs[3 * _RING:4 * _RING]
        bc = lax.axis_index("s") * _NC + lax.axis_index("c")

        pltpu.sync_copy(xts_hbm.at[bc], idx_all)

        def fire_gather(q, s):
            pltpu.async_copy(
                table_hbm.at[idx_all.at[pl.ds(s * _SB, _SB)]],
                rows[q], gsem[q])

        def fire_wb(q, s):
            pltpu.async_copy(
                trans[q], out_hbm.at[pl.ds(s * _TS, _TS), :, bc, :, :],
                wsem[q])

        for q in range(_RING):
            fire_gather(q, q)

        iota16 = lax.iota(jnp.int32, 16)
        rowv = [iota16 + g * 16 for g in range(_SB // 16)]

        def transpose_scale(q):
            rq, tq = rows[q], trans[q]

            def drbody(dr, carry):
                d8v = jnp.broadcast_to(dr * 8, (16,)).astype(jnp.int32)
                for r in range(8):
                    colv = d8v + r
                    for g in range(_SB // 16):
                        tt, c16 = g // (_BLK // 16), g % (_BLK // 16)
                        vals = plsc.load_gather(rq, [rowv[g], colv])
                        tq[tt, dr, r, pl.ds(c16 * 16, 16)] = vals * _SCALE
                return carry
            lax.fori_loop(0, _D // 8, drbody, 0)

        def outer(h, carry):
            for qi in range(_RING):
                s = h * _RING + qi
                pltpu.make_async_copy(
                    table_hbm.at[idx_all.at[pl.ds(s * _SB, _SB)]],
                    rows[qi], gsem[qi]).wait()

                @pl.when(s >= _RING)
                def _():
                    pltpu.make_async_copy(
                        trans[qi],
                        out_hbm.at[pl.ds(s * _TS, _TS), :, bc, :, :],
                        wsem[qi]).wait()

                transpose_scale(qi)
                fire_wb(qi, s)

                @pl.when(s + _RING < n_super)
                def _():
                    fire_gather(qi, s + _RING)
            return carry

        lax.fori_loop(0, n_super // _RING, outer, 0)

        for q in range(_RING):
            pltpu.make_async_copy(
                trans[q], out_hbm.at[pl.ds(0, _TS), :, bc, :, :],
                wsem[q]).wait()

    return k(table, xts)


def kernel(x, table):
    n_b, n_t = x.shape
    d = table.shape[1]
    xts = (
        x.T.astype(jnp.int32)
        .reshape(n_t // _TS, _TS, _NW, _BLK)
        .transpose(2, 0, 1, 3)
        .reshape(_NW, n_t * _BLK)
    )
    o5 = _embed_lookup(table, xts, n_t, n_b)
    return o5.transpose(2, 4, 0, 1, 3).reshape(n_b, n_t, d)


# speedup vs baseline: 1.7820x; 1.7820x over previous
import functools

import jax
import jax.numpy as jnp
from jax import lax
from jax.experimental import pallas as pl
from jax.experimental.pallas import tpu as pltpu
from jax.experimental.pallas import tpu_sc as plsc

_D = 64
_SCALE = 8.0

_NC = 2
_NS = 16
_NW = _NC * _NS

_BLK = 128
_BC = 2
_SB = _BC * _BLK
_PAD = _BLK + 1
_RING = 2
_UNROLL = 8


@functools.partial(jax.jit, static_argnames=("n_t", "n_b"))
def _embed_lookup(table, x_t, n_t, n_b):
    n_duet = n_b // _SB
    t_per_w = n_t // (_NW // n_duet)
    mesh = plsc.VectorSubcoreMesh(core_axis_name="c", subcore_axis_name="s")

    scratch = [pltpu.VMEM((t_per_w, _SB), jnp.int32)]
    scratch += [pltpu.VMEM((_SB, _D), jnp.float32) for _ in range(_RING)]
    scratch += [pltpu.VMEM((_D // 8, _BC, 8, _PAD), jnp.float32)
                for _ in range(_RING)]
    scratch += [pltpu.SemaphoreType.DMA for _ in range(2 * _RING)]

    @functools.partial(
        pl.kernel,
        out_type=jax.ShapeDtypeStruct((n_t, _D // 8, _NW, 8, _BLK),
                                      jnp.float32),
        mesh=mesh,
        scratch_types=scratch,
        compiler_params=pltpu.CompilerParams(
            use_tc_tiling_on_sc=False, needs_layout_passes=False),
    )
    def k(table_hbm, xt_hbm, out_hbm, idx_all, *bufs):
        rows = bufs[0:_RING]
        trans = bufs[_RING:2 * _RING]
        gsem = bufs[2 * _RING:3 * _RING]
        wsem = bufs[3 * _RING:4 * _RING]
        wid = lax.axis_index("s") * _NC + lax.axis_index("c")
        dd = wid % n_duet
        t0 = (wid // n_duet) * t_per_w

        pltpu.sync_copy(
            xt_hbm.at[pl.ds(t0, t_per_w), pl.ds(dd * _SB, _SB)], idx_all)

        def fire_gather(q, n):
            pltpu.async_copy(table_hbm.at[idx_all.at[n]], rows[q], gsem[q])

        def fire_wb(q, n):
            pltpu.async_copy(
                trans[q].at[:, :, :, pl.ds(0, _BLK)],
                out_hbm.at[t0 + n, :, pl.ds(dd * _BC, _BC), :, :],
                wsem[q])

        def wait_wb(q, n):
            pltpu.make_async_copy(
                trans[q].at[:, :, :, pl.ds(0, _BLK)],
                out_hbm.at[t0 + n, :, pl.ds(dd * _BC, _BC), :, :],
                wsem[q]).wait()

        for q in range(_RING):
            fire_gather(q, q)

        iota16 = lax.iota(jnp.int32, 16)
        drv = [(d16 * 16 + iota16) // 8 for d16 in range(_D // 16)]
        rv = [(d16 * 16 + iota16) % 8 for d16 in range(_D // 16)]
        bclv = [jnp.full((16,), b, jnp.int32) for b in range(_BC)]

        def transpose_scale(q):
            rq, tq = rows[q], trans[q]
            for bcl in range(_BC):

                def cbody(i, carry, _bcl=bcl):
                    cb = i * _UNROLL
                    cbv = jnp.broadcast_to(cb, (16,)).astype(jnp.int32)
                    for u in range(_UNROLL):
                        tok = _bcl * _BLK + cb + u
                        cv = cbv + u
                        for d16 in range(_D // 16):
                            vals = rq[tok, pl.ds(d16 * 16, 16)] * _SCALE
                            plsc.store_scatter(
                                tq, [drv[d16], bclv[_bcl], rv[d16], cv],
                                vals)
                    return carry
                lax.fori_loop(0, _BLK // _UNROLL, cbody, 0)

        def outer(h, carry):
            for qi in range(_RING):
                n = h * _RING + qi
                pltpu.make_async_copy(
                    table_hbm.at[idx_all.at[n]], rows[qi], gsem[qi]).wait()

                @pl.when(n >= _RING)
                def _():
                    wait_wb(qi, n)

                transpose_scale(qi)
                fire_wb(qi, n)

                @pl.when(n + _RING < t_per_w)
                def _():
                    fire_gather(qi, n + _RING)
            return carry

        lax.fori_loop(0, t_per_w // _RING, outer, 0)

        for q in range(_RING):
            wait_wb(q, 0)

    return k(table, x_t)


def kernel(x, table):
    n_b, n_t = x.shape
    d = table.shape[1]
    x_t = x.T.astype(jnp.int32)
    o5 = _embed_lookup(table, x_t, n_t, n_b)
    return o5.transpose(2, 4, 0, 1, 3).reshape(n_b, n_t, d)
